# Initial kernel scaffold; baseline (speedup 1.0000x reference)
import functools

import jax
import jax.numpy as jnp
from jax import lax
from jax.experimental import pallas as pl
from jax.experimental.pallas import tpu as pltpu
from jax.experimental.pallas import tpu_sc as plsc

N = 10000
E = 320000
D = 128
G = 64
DOUT = 10

NBLK = 256
N_PAD = 10240
DUMP = N
NT = 32
CHUNK = 128
NCHUNK = 80
E_PAD = NT * CHUNK * NCHUNK
STRIPE = N_PAD // 16


def _sc_mesh():
    return plsc.VectorSubcoreMesh(core_axis_name="c", subcore_axis_name="s")


def _deg_partials(dst3, ones_rows, zrow16):
    @functools.partial(
        pl.kernel,
        out_type=jax.ShapeDtypeStruct((2, N_PAD, 16), jnp.float32),
        mesh=_sc_mesh(),
        scratch_types=[
            pltpu.VMEM((NCHUNK, CHUNK), jnp.int32),
            pltpu.VMEM((CHUNK, 16), jnp.float32),
            pltpu.VMEM_SHARED((N_PAD, 16), jnp.float32),
        ],
    )
    def k(dst_hbm, ones_hbm, z_hbm, out_hbm, dstv, onesv, acc):
        c = lax.axis_index("c")
        s = lax.axis_index("s")
        wid = c * 16 + s
        pltpu.sync_copy(z_hbm, acc.at[pl.ds(s * STRIPE, STRIPE)])
        pltpu.sync_copy(ones_hbm, onesv)
        pltpu.sync_copy(dst_hbm.at[wid], dstv)
        plsc.subcore_barrier()

        @pl.loop(0, NCHUNK)
        def _(j):
            pltpu.sync_copy(onesv, acc.at[dstv.at[j]], add=True)

        plsc.subcore_barrier()
        pltpu.sync_copy(acc.at[pl.ds(s * STRIPE, STRIPE)],
                        out_hbm.at[c, pl.ds(s * STRIPE, STRIPE)])

    return k(dst3, ones_rows, zrow16)


def _agg_partials(g, src3, dst3, zrow):
    @functools.partial(
        pl.kernel,
        out_type=jax.ShapeDtypeStruct((2, N_PAD, D), jnp.float32),
        mesh=_sc_mesh(),
        scratch_types=[
            pltpu.VMEM((NCHUNK, CHUNK), jnp.int32),
            pltpu.VMEM((NCHUNK, CHUNK), jnp.int32),
            pltpu.VMEM((CHUNK, D), jnp.float32),
            pltpu.VMEM((CHUNK, D), jnp.float32),
            pltpu.VMEM_SHARED((N_PAD, D), jnp.float32),
            pltpu.SemaphoreType.DMA,
            pltpu.SemaphoreType.DMA,
        ],
    )
    def k(g_hbm, src_hbm, dst_hbm, z_hbm, out_hbm,
          srcv, dstv, bufa, bufb, acc, sema, semb):
        c = lax.axis_index("c")
        s = lax.axis_index("s")
        wid = c * 16 + s
        pltpu.sync_copy(z_hbm, acc.at[pl.ds(s * STRIPE, STRIPE)])
        pltpu.sync_copy(src_hbm.at[wid], srcv)
        pltpu.sync_copy(dst_hbm.at[wid], dstv)
        plsc.subcore_barrier()

        pltpu.async_copy(g_hbm.at[srcv.at[0]], bufa, sema)
        pltpu.async_copy(g_hbm.at[srcv.at[1]], bufb, semb)

        @pl.loop(0, NCHUNK, step=2)
        def _(j):
            pltpu.make_async_copy(g_hbm.at[srcv.at[j]], bufa, sema).wait()
            pltpu.sync_copy(bufa, acc.at[dstv.at[j]], add=True)

            @pl.when(j + 2 < NCHUNK)
            def _():
                pltpu.async_copy(g_hbm.at[srcv.at[j + 2]], bufa, sema)

            pltpu.make_async_copy(g_hbm.at[srcv.at[j + 1]], bufb, semb).wait()
            pltpu.sync_copy(bufb, acc.at[dstv.at[j + 1]], add=True)

            @pl.when(j + 3 < NCHUNK)
            def _():
                pltpu.async_copy(g_hbm.at[srcv.at[j + 3]], bufb, semb)

        plsc.subcore_barrier()
        pltpu.sync_copy(acc.at[pl.ds(s * STRIPE, STRIPE)],
                        out_hbm.at[c, pl.ds(s * STRIPE, STRIPE)])

    return k(g, src3, dst3, zrow)


def _dinv_block(d_ref):
    deg = d_ref[0, :, 0:1] + d_ref[1, :, 0:1] + 1.0
    return lax.rsqrt(deg)


def _tc_first(x_p, W0, degp):
    def body(x_ref, w_ref, d_ref, g_ref):
        dinv = _dinv_block(d_ref)
        g_ref[...] = dinv * jnp.dot(x_ref[...], w_ref[...],
                                    preferred_element_type=jnp.float32)

    return pl.pallas_call(
        body,
        grid=(N_PAD // NBLK,),
        in_specs=[
            pl.BlockSpec((NBLK, D), lambda i: (i, 0)),
            pl.BlockSpec((D, D), lambda i: (0, 0)),
            pl.BlockSpec((2, NBLK, 16), lambda i: (0, i, 0)),
        ],
        out_specs=pl.BlockSpec((NBLK, D), lambda i: (i, 0)),
        out_shape=jax.ShapeDtypeStruct((N_PAD, D), jnp.float32),
    )(x_p, W0, degp)


def _tc_layer(aggp, g_prev, degp, bias, W_next):
    def body(a_ref, g_ref, d_ref, b_ref, w_ref, o_ref):
        dinv = _dinv_block(d_ref)
        h = dinv * (a_ref[0] + a_ref[1] + g_ref[...]) + b_ref[...]
        h = jnp.maximum(h, 0.0)
        o_ref[...] = dinv * jnp.dot(h, w_ref[...],
                                    preferred_element_type=jnp.float32)

    return pl.pallas_call(
        body,
        grid=(N_PAD // NBLK,),
        in_specs=[
            pl.BlockSpec((2, NBLK, D), lambda i: (0, i, 0)),
            pl.BlockSpec((NBLK, D), lambda i: (i, 0)),
            pl.BlockSpec((2, NBLK, 16), lambda i: (0, i, 0)),
            pl.BlockSpec((1, D), lambda i: (0, 0)),
            pl.BlockSpec((D, D), lambda i: (0, 0)),
        ],
        out_specs=pl.BlockSpec((NBLK, D), lambda i: (i, 0)),
        out_shape=jax.ShapeDtypeStruct((N_PAD, D), jnp.float32),
    )(aggp, g_prev, degp, bias, W_next)


def _tc_final(aggp, g_prev, degp, bias, batch3, Wlin, blin):
    nsteps = N_PAD // NBLK

    def body(a_ref, g_ref, d_ref, b_ref, bt_ref, wl_ref, bl_ref,
             o_ref, pooled, counts):
        i = pl.program_id(0)

        @pl.when(i == 0)
        def _():
            pooled[...] = jnp.zeros_like(pooled)
            counts[...] = jnp.zeros_like(counts)

        dinv = _dinv_block(d_ref)
        h = dinv * (a_ref[0] + a_ref[1] + g_ref[...]) + b_ref[...]
        h = jnp.maximum(h, 0.0)

        bvals = bt_ref[0]
        gids = lax.broadcasted_iota(jnp.int32, (G, 1), 0)
        onehot = (gids == bvals).astype(jnp.float32)
        pooled[...] += jnp.dot(onehot, h, preferred_element_type=jnp.float32)
        counts[...] += jnp.sum(onehot, axis=1, keepdims=True)

        @pl.when(i == nsteps - 1)
        def _():
            mean = pooled[...] / jnp.maximum(counts[:, 0:1], 1.0)
            o_ref[...] = jnp.dot(mean, wl_ref[...],
                                 preferred_element_type=jnp.float32) + bl_ref[...]

    return pl.pallas_call(
        body,
        grid=(nsteps,),
        in_specs=[
            pl.BlockSpec((2, NBLK, D), lambda i: (0, i, 0)),
            pl.BlockSpec((NBLK, D), lambda i: (i, 0)),
            pl.BlockSpec((2, NBLK, 16), lambda i: (0, i, 0)),
            pl.BlockSpec((1, D), lambda i: (0, 0)),
            pl.BlockSpec((1, 1, NBLK), lambda i: (i, 0, 0)),
            pl.BlockSpec((D, DOUT), lambda i: (0, 0)),
            pl.BlockSpec((1, DOUT), lambda i: (0, 0)),
        ],
        out_specs=pl.BlockSpec((G, DOUT), lambda i: (0, 0)),
        out_shape=jax.ShapeDtypeStruct((G, DOUT), jnp.float32),
        scratch_shapes=[
            pltpu.VMEM((G, D), jnp.float32),
            pltpu.VMEM((G, 1), jnp.float32),
        ],
    )(aggp, g_prev, degp, bias, batch3, Wlin, blin)


def kernel(x, edge_index, batch, W0, b0, W, b, Wlin, blin):
    f32 = jnp.float32
    x_p = jnp.zeros((N_PAD, D), f32).at[:N].set(x)

    src = edge_index[0].astype(jnp.int32)
    dst = edge_index[1].astype(jnp.int32)
    padlen = E_PAD - E
    pad_idx = jnp.full((padlen,), DUMP, jnp.int32)
    src3 = jnp.concatenate([src, pad_idx]).reshape(NT, NCHUNK, CHUNK)
    dst3 = jnp.concatenate([dst, pad_idx]).reshape(NT, NCHUNK, CHUNK)

    batch_p = jnp.full((N_PAD,), G, jnp.int32).at[:N].set(batch.astype(jnp.int32))
    batch3 = batch_p.reshape(N_PAD // NBLK, 1, NBLK)

    ones_rows = jnp.ones((CHUNK, 16), f32)
    zrow16 = jnp.zeros((STRIPE, 16), f32)
    zrow = jnp.zeros((STRIPE, D), f32)
    b0_2d = b0.reshape(1, D).astype(f32)
    b_2d = b.reshape(1, D).astype(f32)
    blin_2d = blin.reshape(1, DOUT).astype(f32)

    degp = _deg_partials(dst3, ones_rows, zrow16)

    g1 = _tc_first(x_p, W0.astype(f32), degp)
    agg1 = _agg_partials(g1, src3, dst3, zrow)
    g2 = _tc_layer(agg1, g1, degp, b0_2d, W.astype(f32))
    agg2 = _agg_partials(g2, src3, dst3, zrow)
    g3 = _tc_layer(agg2, g2, degp, b_2d, W.astype(f32))
    agg3 = _agg_partials(g3, src3, dst3, zrow)

    return _tc_final(agg3, g3, degp, b_2d, batch3, Wlin.astype(f32), blin_2d)


# speedup vs baseline: 8.1224x; 8.1224x over previous
import functools

import jax
import jax.numpy as jnp
from jax import lax
from jax.experimental import pallas as pl
from jax.experimental.pallas import tpu as pltpu
from jax.experimental.pallas import tpu_sc as plsc

N = 10000
E = 320000
D = 128
G = 64
DOUT = 10

NBLK = 256
N_PAD = 10240
DUMP = N
NT = 32
CHUNK = 128
NCHUNK = 80
E_PAD = NT * CHUNK * NCHUNK
IP = 2
NCH_P = NCHUNK // IP
STRIPE = N_PAD // 16


def _sc_mesh():
    return plsc.VectorSubcoreMesh(core_axis_name="c", subcore_axis_name="s")


def _deg_partials(dst3, ones_rows, zrow):
    @functools.partial(
        pl.kernel,
        out_type=jax.ShapeDtypeStruct((2, N_PAD, D), jnp.float32),
        mesh=_sc_mesh(),
        scratch_types=[
            pltpu.VMEM((NCH_P, CHUNK), jnp.int32),
            pltpu.VMEM((CHUNK, D), jnp.float32),
            pltpu.VMEM_SHARED((N_PAD, D), jnp.float32),
        ],
    )
    def k(dst_hbm, ones_hbm, z_hbm, out_hbm, dstv, onesv, acc):
        c = lax.axis_index("c")
        s = lax.axis_index("s")
        wid = c * 16 + s
        pltpu.sync_copy(z_hbm, acc.at[pl.ds(s * STRIPE, STRIPE)])
        pltpu.sync_copy(ones_hbm, onesv)
        plsc.subcore_barrier()

        for p in range(IP):
            pltpu.sync_copy(dst_hbm.at[wid, pl.ds(p * NCH_P, NCH_P)], dstv)

            @pl.loop(0, NCH_P)
            def _(j):
                pltpu.sync_copy(onesv, acc.at[dstv.at[j]], add=True)

        plsc.subcore_barrier()
        pltpu.sync_copy(acc.at[pl.ds(s * STRIPE, STRIPE)],
                        out_hbm.at[c, pl.ds(s * STRIPE, STRIPE)])

    return k(dst3, ones_rows, zrow)


def _agg_partials(g, src3, dst3, zrow):
    @functools.partial(
        pl.kernel,
        out_type=jax.ShapeDtypeStruct((2, N_PAD, D), jnp.float32),
        mesh=_sc_mesh(),
        scratch_types=[
            pltpu.VMEM((NCH_P, CHUNK), jnp.int32),
            pltpu.VMEM((NCH_P, CHUNK), jnp.int32),
            pltpu.VMEM((CHUNK, D), jnp.float32),
            pltpu.VMEM((CHUNK, D), jnp.float32),
            pltpu.VMEM_SHARED((N_PAD, D), jnp.float32),
            pltpu.SemaphoreType.DMA,
            pltpu.SemaphoreType.DMA,
        ],
    )
    def k(g_hbm, src_hbm, dst_hbm, z_hbm, out_hbm,
          srcv, dstv, bufa, bufb, acc, sema, semb):
        c = lax.axis_index("c")
        s = lax.axis_index("s")
        wid = c * 16 + s
        pltpu.sync_copy(z_hbm, acc.at[pl.ds(s * STRIPE, STRIPE)])
        plsc.subcore_barrier()

        for p in range(IP):
            pltpu.sync_copy(src_hbm.at[wid, pl.ds(p * NCH_P, NCH_P)], srcv)
            pltpu.sync_copy(dst_hbm.at[wid, pl.ds(p * NCH_P, NCH_P)], dstv)

            pltpu.async_copy(g_hbm.at[srcv.at[0]], bufa, sema)
            pltpu.async_copy(g_hbm.at[srcv.at[1]], bufb, semb)

            @pl.loop(0, NCH_P, step=2)
            def _(j):
                pltpu.make_async_copy(g_hbm.at[srcv.at[j]], bufa, sema).wait()
                pltpu.sync_copy(bufa, acc.at[dstv.at[j]], add=True)

                @pl.when(j + 2 < NCH_P)
                def _():
                    pltpu.async_copy(g_hbm.at[srcv.at[j + 2]], bufa, sema)

                pltpu.make_async_copy(g_hbm.at[srcv.at[j + 1]], bufb, semb).wait()
                pltpu.sync_copy(bufb, acc.at[dstv.at[j + 1]], add=True)

                @pl.when(j + 3 < NCH_P)
                def _():
                    pltpu.async_copy(g_hbm.at[srcv.at[j + 3]], bufb, semb)

        plsc.subcore_barrier()
        pltpu.sync_copy(acc.at[pl.ds(s * STRIPE, STRIPE)],
                        out_hbm.at[c, pl.ds(s * STRIPE, STRIPE)])

    return k(g, src3, dst3, zrow)


def _dinv_block(d_ref):
    deg = d_ref[0, :, 0:1] + d_ref[1, :, 0:1] + 1.0
    return lax.rsqrt(deg)


def _tc_first(x_p, W0, degp):
    def body(x_ref, w_ref, d_ref, g_ref):
        dinv = _dinv_block(d_ref)
        g_ref[...] = dinv * jnp.dot(x_ref[...], w_ref[...],
                                    preferred_element_type=jnp.float32)

    return pl.pallas_call(
        body,
        grid=(N_PAD // NBLK,),
        in_specs=[
            pl.BlockSpec((NBLK, D), lambda i: (i, 0)),
            pl.BlockSpec((D, D), lambda i: (0, 0)),
            pl.BlockSpec((2, NBLK, D), lambda i: (0, i, 0)),
        ],
        out_specs=pl.BlockSpec((NBLK, D), lambda i: (i, 0)),
        out_shape=jax.ShapeDtypeStruct((N_PAD, D), jnp.float32),
    )(x_p, W0, degp)


def _tc_layer(aggp, g_prev, degp, bias, W_next):
    def body(a_ref, g_ref, d_ref, b_ref, w_ref, o_ref):
        dinv = _dinv_block(d_ref)
        h = dinv * (a_ref[0] + a_ref[1] + g_ref[...]) + b_ref[...]
        h = jnp.maximum(h, 0.0)
        o_ref[...] = dinv * jnp.dot(h, w_ref[...],
                                    preferred_element_type=jnp.float32)

    return pl.pallas_call(
        body,
        grid=(N_PAD // NBLK,),
        in_specs=[
            pl.BlockSpec((2, NBLK, D), lambda i: (0, i, 0)),
            pl.BlockSpec((NBLK, D), lambda i: (i, 0)),
            pl.BlockSpec((2, NBLK, D), lambda i: (0, i, 0)),
            pl.BlockSpec((1, D), lambda i: (0, 0)),
            pl.BlockSpec((D, D), lambda i: (0, 0)),
        ],
        out_specs=pl.BlockSpec((NBLK, D), lambda i: (i, 0)),
        out_shape=jax.ShapeDtypeStruct((N_PAD, D), jnp.float32),
    )(aggp, g_prev, degp, bias, W_next)


def _tc_final(aggp, g_prev, degp, bias, batch3, Wlin, blin):
    nsteps = N_PAD // NBLK

    def body(a_ref, g_ref, d_ref, b_ref, bt_ref, wl_ref, bl_ref,
             o_ref, pooled, counts):
        i = pl.program_id(0)

        @pl.when(i == 0)
        def _():
            pooled[...] = jnp.zeros_like(pooled)
            counts[...] = jnp.zeros_like(counts)

        dinv = _dinv_block(d_ref)
        h = dinv * (a_ref[0] + a_ref[1] + g_ref[...]) + b_ref[...]
        h = jnp.maximum(h, 0.0)

        bvals = bt_ref[0]
        gids = lax.broadcasted_iota(jnp.int32, (G, 1), 0)
        onehot = (gids == bvals).astype(jnp.float32)
        pooled[...] += jnp.dot(onehot, h, preferred_element_type=jnp.float32)
        counts[...] += jnp.sum(onehot, axis=1, keepdims=True)

        @pl.when(i == nsteps - 1)
        def _():
            mean = pooled[...] / jnp.maximum(counts[:, 0:1], 1.0)
            o_ref[...] = jnp.dot(mean, wl_ref[...],
                                 preferred_element_type=jnp.float32) + bl_ref[...]

    return pl.pallas_call(
        body,
        grid=(nsteps,),
        in_specs=[
            pl.BlockSpec((2, NBLK, D), lambda i: (0, i, 0)),
            pl.BlockSpec((NBLK, D), lambda i: (i, 0)),
            pl.BlockSpec((2, NBLK, D), lambda i: (0, i, 0)),
            pl.BlockSpec((1, D), lambda i: (0, 0)),
            pl.BlockSpec((1, 1, NBLK), lambda i: (i, 0, 0)),
            pl.BlockSpec((D, DOUT), lambda i: (0, 0)),
            pl.BlockSpec((1, DOUT), lambda i: (0, 0)),
        ],
        out_specs=pl.BlockSpec((G, DOUT), lambda i: (0, 0)),
        out_shape=jax.ShapeDtypeStruct((G, DOUT), jnp.float32),
        scratch_shapes=[
            pltpu.VMEM((G, D), jnp.float32),
            pltpu.VMEM((G, 1), jnp.float32),
        ],
    )(aggp, g_prev, degp, bias, batch3, Wlin, blin)


def kernel(x, edge_index, batch, W0, b0, W, b, Wlin, blin):
    f32 = jnp.float32
    x_p = jnp.zeros((N_PAD, D), f32).at[:N].set(x)

    src = edge_index[0].astype(jnp.int32)
    dst = edge_index[1].astype(jnp.int32)
    padlen = E_PAD - E
    pad_idx = jnp.full((padlen,), DUMP, jnp.int32)
    src3 = jnp.concatenate([src, pad_idx]).reshape(NT, NCHUNK, CHUNK)
    dst3 = jnp.concatenate([dst, pad_idx]).reshape(NT, NCHUNK, CHUNK)

    batch_p = jnp.full((N_PAD,), G, jnp.int32).at[:N].set(batch.astype(jnp.int32))
    batch3 = batch_p.reshape(N_PAD // NBLK, 1, NBLK)

    ones_rows = jnp.ones((CHUNK, D), f32)
    zrow = jnp.zeros((STRIPE, D), f32)
    b0_2d = b0.reshape(1, D).astype(f32)
    b_2d = b.reshape(1, D).astype(f32)
    blin_2d = blin.reshape(1, DOUT).astype(f32)

    degp = _deg_partials(dst3, ones_rows, zrow)

    g1 = _tc_first(x_p, W0.astype(f32), degp)
    agg1 = _agg_partials(g1, src3, dst3, zrow)
    g2 = _tc_layer(agg1, g1, degp, b0_2d, W.astype(f32))
    agg2 = _agg_partials(g2, src3, dst3, zrow)
    g3 = _tc_layer(agg2, g2, degp, b_2d, W.astype(f32))
    agg3 = _agg_partials(g3, src3, dst3, zrow)

    return _tc_final(agg3, g3, degp, b_2d, batch3, Wlin.astype(f32), blin_2d)


# speedup vs baseline: 8.1233x; 1.0001x over previous
import functools

import jax
import jax.numpy as jnp
from jax import lax
from jax.experimental import pallas as pl
from jax.experimental.pallas import tpu as pltpu
from jax.experimental.pallas import tpu_sc as plsc

N = 10000
E = 320000
D = 128
G = 64
DOUT = 10

NBLK = 256
N_PAD = 10240
DUMP = N
NT = 32
CHUNK = 128
NCHUNK = 80
E_PAD = NT * CHUNK * NCHUNK
IP = 2
NCH_P = NCHUNK // IP
STRIPE = N_PAD // 16


def _sc_mesh():
    return plsc.VectorSubcoreMesh(core_axis_name="c", subcore_axis_name="s")


def _deg_partials(dst3, ones_rows, zrow):
    @functools.partial(
        pl.kernel,
        out_type=jax.ShapeDtypeStruct((2, N_PAD, D), jnp.float32),
        mesh=_sc_mesh(),
        scratch_types=[
            pltpu.VMEM((NCH_P, CHUNK), jnp.int32),
            pltpu.VMEM((CHUNK, D), jnp.float32),
            pltpu.VMEM_SHARED((N_PAD, D), jnp.float32),
        ],
    )
    def k(dst_hbm, ones_hbm, z_hbm, out_hbm, dstv, onesv, acc):
        c = lax.axis_index("c")
        s = lax.axis_index("s")
        wid = c * 16 + s
        pltpu.sync_copy(z_hbm, acc.at[pl.ds(s * STRIPE, STRIPE)])
        pltpu.sync_copy(ones_hbm, onesv)
        plsc.subcore_barrier()

        for p in range(IP):
            pltpu.sync_copy(dst_hbm.at[wid, pl.ds(p * NCH_P, NCH_P)], dstv)

            @pl.loop(0, NCH_P)
            def _(j):
                pltpu.sync_copy(onesv, acc.at[dstv.at[j]], add=True)

        plsc.subcore_barrier()
        pltpu.sync_copy(acc.at[pl.ds(s * STRIPE, STRIPE)],
                        out_hbm.at[c, pl.ds(s * STRIPE, STRIPE)])

    return k(dst3, ones_rows, zrow)


def _agg_partials(g, src3, dst3, zrow):
    @functools.partial(
        pl.kernel,
        out_type=jax.ShapeDtypeStruct((2, N_PAD, D), jnp.float32),
        mesh=_sc_mesh(),
        scratch_types=[
            pltpu.VMEM((NCH_P, CHUNK), jnp.int32),
            pltpu.VMEM((NCH_P, CHUNK), jnp.int32),
            pltpu.VMEM((CHUNK, D), jnp.float32),
            pltpu.VMEM((CHUNK, D), jnp.float32),
            pltpu.VMEM_SHARED((N_PAD, D), jnp.float32),
            pltpu.SemaphoreType.DMA,
            pltpu.SemaphoreType.DMA,
        ],
    )
    def k(g_hbm, src_hbm, dst_hbm, z_hbm, out_hbm,
          srcv, dstv, bufa, bufb, acc, sema, semb):
        c = lax.axis_index("c")
        s = lax.axis_index("s")
        wid = c * 16 + s
        pltpu.sync_copy(z_hbm, acc.at[pl.ds(s * STRIPE, STRIPE)])
        plsc.subcore_barrier()

        for p in range(IP):
            pltpu.sync_copy(src_hbm.at[wid, pl.ds(p * NCH_P, NCH_P)], srcv)
            pltpu.sync_copy(dst_hbm.at[wid, pl.ds(p * NCH_P, NCH_P)], dstv)

            pltpu.async_copy(g_hbm.at[srcv.at[0]], bufa, sema)
            pltpu.async_copy(g_hbm.at[srcv.at[1]], bufb, semb)

            @pl.loop(0, NCH_P, step=2)
            def _(j):
                pltpu.make_async_copy(g_hbm.at[srcv.at[j]], bufa, sema).wait()
                pltpu.sync_copy(bufa, acc.at[dstv.at[j]], add=True)

                @pl.when(j + 2 < NCH_P)
                def _():
                    pltpu.async_copy(g_hbm.at[srcv.at[j + 2]], bufa, sema)

                pltpu.make_async_copy(g_hbm.at[srcv.at[j + 1]], bufb, semb).wait()
                pltpu.sync_copy(bufb, acc.at[dstv.at[j + 1]], add=True)

                @pl.when(j + 3 < NCH_P)
                def _():
                    pltpu.async_copy(g_hbm.at[srcv.at[j + 3]], bufb, semb)

        plsc.subcore_barrier()
        pltpu.sync_copy(acc.at[pl.ds(s * STRIPE, STRIPE)],
                        out_hbm.at[c, pl.ds(s * STRIPE, STRIPE)])

    return k(g, src3, dst3, zrow)


def _dinv_block(d_ref):
    deg = d_ref[0, :, 0:1] + d_ref[1, :, 0:1] + 1.0
    return lax.rsqrt(deg)


def _tc_first(x_p, W0, degp):
    def body(x_ref, w_ref, d_ref, g_ref):
        dinv = _dinv_block(d_ref)
        g_ref[...] = dinv * jnp.dot(x_ref[...], w_ref[...],
                                    preferred_element_type=jnp.float32)

    return pl.pallas_call(
        body,
        grid=(N_PAD // NBLK,),
        in_specs=[
            pl.BlockSpec((NBLK, D), lambda i: (i, 0)),
            pl.BlockSpec((D, D), lambda i: (0, 0)),
            pl.BlockSpec((2, NBLK, D), lambda i: (0, i, 0)),
        ],
        out_specs=pl.BlockSpec((NBLK, D), lambda i: (i, 0)),
        out_shape=jax.ShapeDtypeStruct((N_PAD, D), jnp.float32),
    )(x_p, W0, degp)


def _tc_layer(aggp, g_prev, degp, bias, W_next):
    def body(a_ref, g_ref, d_ref, b_ref, w_ref, o_ref):
        dinv = _dinv_block(d_ref)
        h = dinv * (a_ref[0] + a_ref[1] + g_ref[...]) + b_ref[...]
        h = jnp.maximum(h, 0.0)
        o_ref[...] = dinv * jnp.dot(h, w_ref[...],
                                    preferred_element_type=jnp.float32)

    return pl.pallas_call(
        body,
        grid=(N_PAD // NBLK,),
        in_specs=[
            pl.BlockSpec((2, NBLK, D), lambda i: (0, i, 0)),
            pl.BlockSpec((NBLK, D), lambda i: (i, 0)),
            pl.BlockSpec((2, NBLK, D), lambda i: (0, i, 0)),
            pl.BlockSpec((1, D), lambda i: (0, 0)),
            pl.BlockSpec((D, D), lambda i: (0, 0)),
        ],
        out_specs=pl.BlockSpec((NBLK, D), lambda i: (i, 0)),
        out_shape=jax.ShapeDtypeStruct((N_PAD, D), jnp.float32),
    )(aggp, g_prev, degp, bias, W_next)


def _tc_final(aggp, g_prev, degp, bias, batch3, Wlin, blin):
    nsteps = N_PAD // NBLK

    def body(a_ref, g_ref, d_ref, b_ref, bt_ref, wl_ref, bl_ref,
             o_ref, pooled, counts):
        i = pl.program_id(0)

        @pl.when(i == 0)
        def _():
            pooled[...] = jnp.zeros_like(pooled)
            counts[...] = jnp.zeros_like(counts)

        dinv = _dinv_block(d_ref)
        h = dinv * (a_ref[0] + a_ref[1] + g_ref[...]) + b_ref[...]
        h = jnp.maximum(h, 0.0)

        bvals = bt_ref[0]
        gids = lax.broadcasted_iota(jnp.int32, (G, 1), 0)
        onehot = (gids == bvals).astype(jnp.float32)
        pooled[...] += jnp.dot(onehot, h, preferred_element_type=jnp.float32)
        counts[...] += jnp.sum(onehot, axis=1, keepdims=True)

        @pl.when(i == nsteps - 1)
        def _():
            mean = pooled[...] / jnp.maximum(counts[:, 0:1], 1.0)
            o_ref[...] = jnp.dot(mean, wl_ref[...],
                                 preferred_element_type=jnp.float32) + bl_ref[...]

    return pl.pallas_call(
        body,
        grid=(nsteps,),
        in_specs=[
            pl.BlockSpec((2, NBLK, D), lambda i: (0, i, 0)),
            pl.BlockSpec((NBLK, D), lambda i: (i, 0)),
            pl.BlockSpec((2, NBLK, D), lambda i: (0, i, 0)),
            pl.BlockSpec((1, D), lambda i: (0, 0)),
            pl.BlockSpec((1, 1, NBLK), lambda i: (i, 0, 0)),
            pl.BlockSpec((D, DOUT), lambda i: (0, 0)),
            pl.BlockSpec((1, DOUT), lambda i: (0, 0)),
        ],
        out_specs=pl.BlockSpec((G, DOUT), lambda i: (0, 0)),
        out_shape=jax.ShapeDtypeStruct((G, DOUT), jnp.float32),
        scratch_shapes=[
            pltpu.VMEM((G, D), jnp.float32),
            pltpu.VMEM((G, 1), jnp.float32),
        ],
    )(aggp, g_prev, degp, bias, batch3, Wlin, blin)


def kernel(x, edge_index, batch, W0, b0, W, b, Wlin, blin):
    f32 = jnp.float32
    x_p = jnp.zeros((N_PAD, D), f32).at[:N].set(x)

    src = edge_index[0].astype(jnp.int32)
    dst = edge_index[1].astype(jnp.int32)
    padlen = E_PAD - E
    pad_src = jnp.full((padlen,), DUMP, jnp.int32)
    pad_dst = DUMP + (jnp.arange(padlen, dtype=jnp.int32) % (N_PAD - N))
    src3 = jnp.concatenate([src, pad_src]).reshape(NT, NCHUNK, CHUNK)
    dst3 = jnp.concatenate([dst, pad_dst]).reshape(NT, NCHUNK, CHUNK)

    batch_p = jnp.full((N_PAD,), G, jnp.int32).at[:N].set(batch.astype(jnp.int32))
    batch3 = batch_p.reshape(N_PAD // NBLK, 1, NBLK)

    ones_rows = jnp.ones((CHUNK, D), f32)
    zrow = jnp.zeros((STRIPE, D), f32)
    b0_2d = b0.reshape(1, D).astype(f32)
    b_2d = b.reshape(1, D).astype(f32)
    blin_2d = blin.reshape(1, DOUT).astype(f32)

    degp = _deg_partials(dst3, ones_rows, zrow)

    g1 = _tc_first(x_p, W0.astype(f32), degp)
    agg1 = _agg_partials(g1, src3, dst3, zrow)
    g2 = _tc_layer(agg1, g1, degp, b0_2d, W.astype(f32))
    agg2 = _agg_partials(g2, src3, dst3, zrow)
    g3 = _tc_layer(agg2, g2, degp, b_2d, W.astype(f32))
    agg3 = _agg_partials(g3, src3, dst3, zrow)

    return _tc_final(agg3, g3, degp, b_2d, batch3, Wlin.astype(f32), blin_2d)
